# baseline (speedup 1.0000x reference)
import functools

import jax
import jax.numpy as jnp
from jax import lax
from jax.experimental import pallas as pl
from jax.experimental.pallas import tpu as pltpu
from jax.experimental.pallas import tpu_sc as plsc

NE = 16
NK = 2
DM = 1024
DF = 4096
NT = 4096
CAP = 2048
NP = NT * NK
NROWS = NE * CAP

FC = 512
NF = DF // FC

NW = 32
DCH = 64
DNCH = (NROWS // NW) // DCH
CT = 32
CNCH = (NT // NW) // CT
CROWS = NK * CT


def _dispatch_body(x_hbm, src_hbm, out_hbm, idx_v, rows_v, sem):
    wid = lax.axis_index("s") * 2 + lax.axis_index("c")
    pltpu.sync_copy(src_hbm.at[wid], idx_v)

    def chunk(i, carry):
        pltpu.async_copy(x_hbm.at[idx_v.at[i]], rows_v, sem).wait()
        pltpu.sync_copy(rows_v, out_hbm.at[pl.ds(wid * (DNCH * DCH) + i * DCH, DCH)])
        return carry

    lax.fori_loop(0, DNCH, chunk, 0)


def _combine_body(y_hbm, ridx_hbm, out_hbm, idx_v, rows_v, out_v, sem):
    wid = lax.axis_index("s") * 2 + lax.axis_index("c")
    pltpu.sync_copy(ridx_hbm.at[wid], idx_v)

    def chunk(i, carry):
        pltpu.async_copy(y_hbm.at[idx_v.at[i]], rows_v, sem).wait()

        def tok(t, c2):
            def dpart(j, c3):
                s = pl.ds(j * 16, 16)
                out_v[t, s] = rows_v[2 * t, s] + rows_v[2 * t + 1, s]
                return c3
            return lax.fori_loop(0, DM // 16, dpart, c2)

        lax.fori_loop(0, CT, tok, 0)
        pltpu.sync_copy(out_v, out_hbm.at[pl.ds(wid * (CNCH * CT) + i * CT, CT)])
        return carry

    lax.fori_loop(0, CNCH, chunk, 0)


@functools.cache
def _make_sc_kernels():
    mesh = plsc.VectorSubcoreMesh(core_axis_name="c", subcore_axis_name="s")
    dispatch = functools.partial(
        pl.kernel,
        out_type=jax.ShapeDtypeStruct((NROWS, DM), jnp.float32),
        mesh=mesh,
        scratch_types=[
            pltpu.VMEM((DNCH, DCH), jnp.int32),
            pltpu.VMEM((DCH, DM), jnp.float32),
            pltpu.SemaphoreType.DMA,
        ],
    )(_dispatch_body)
    combine = functools.partial(
        pl.kernel,
        out_type=jax.ShapeDtypeStruct((NT, DM), jnp.float32),
        mesh=mesh,
        scratch_types=[
            pltpu.VMEM((CNCH, CROWS), jnp.int32),
            pltpu.VMEM((CROWS, DM), jnp.float32),
            pltpu.VMEM((CT, DM), jnp.float32),
            pltpu.SemaphoreType.DMA,
        ],
    )(_combine_body)
    return dispatch, combine


def _mlp_body(gs_ref, xb_ref, w1_ref, w2_ref, out_ref):
    e = pl.program_id(0)
    f = pl.program_id(1)

    @pl.when(f == 0)
    def _init():
        out_ref[...] = jnp.zeros_like(out_ref)

    @pl.when(e < NE)
    def _compute():
        a = xb_ref[...]
        h = jax.nn.gelu(
            lax.dot_general(a, w1_ref[0], (((1,), (1,)), ((), ())),
                            preferred_element_type=jnp.float32))
        out_ref[...] += lax.dot_general(
            h, w2_ref[0], (((1,), (1,)), ((), ())),
            preferred_element_type=jnp.float32)

    @pl.when(f == NF - 1)
    def _finish():
        out_ref[...] = out_ref[...] * gs_ref[...]


_mlp_call = pl.pallas_call(
    _mlp_body,
    grid=(NE + 1, NF),
    in_specs=[
        pl.BlockSpec((CAP, 1), lambda e, f: (e, 0)),
        pl.BlockSpec((CAP, DM), lambda e, f: (jnp.minimum(e, NE - 1), 0)),
        pl.BlockSpec((1, FC, DM), lambda e, f: (jnp.minimum(e, NE - 1), f, 0)),
        pl.BlockSpec((1, DM, FC), lambda e, f: (jnp.minimum(e, NE - 1), 0, f)),
    ],
    out_specs=pl.BlockSpec((CAP, DM), lambda e, f: (e, 0)),
    out_shape=jax.ShapeDtypeStruct(((NE + 1) * CAP, DM), jnp.float32),
    compiler_params=pltpu.CompilerParams(
        dimension_semantics=("parallel", "arbitrary")),
)


def _routing(gate_weight, choosed_experts):
    flat_e = choosed_experts.reshape(NP)
    order = jnp.argsort(flat_e, stable=True).astype(jnp.int32)
    sorted_e = flat_e[order]
    counts = jnp.bincount(flat_e, length=NE).astype(jnp.int32)
    starts = jnp.concatenate(
        [jnp.zeros((1,), jnp.int32), jnp.cumsum(counts)[:-1].astype(jnp.int32)])
    c_idx = jnp.arange(CAP, dtype=jnp.int32)
    gidx = jnp.clip(starts[:, None] + c_idx[None, :], 0, NP - 1)
    src_pair = order[gidx]
    src_tok = (src_pair // NK).astype(jnp.int32)
    valid = c_idx[None, :] < jnp.minimum(counts, CAP)[:, None]
    gate_slot = jnp.where(valid, gate_weight.reshape(NP)[src_pair], 0.0)
    gate_pad = jnp.concatenate(
        [gate_slot.reshape(NROWS, 1),
         jnp.zeros((CAP, 1), jnp.float32)], axis=0)
    rank = jnp.arange(NP, dtype=jnp.int32) - starts[sorted_e]
    pos = jnp.zeros((NP,), jnp.int32).at[order].set(rank)
    rowidx = jnp.where(pos < CAP, flat_e * CAP + pos, NROWS).astype(jnp.int32)
    return src_tok, gate_pad, rowidx


def kernel(x, gate_weight, choosed_experts, W1, W2):
    dispatch, combine = _make_sc_kernels()
    src_tok, gate_pad, rowidx = _routing(gate_weight, choosed_experts)
    disp = dispatch(x, src_tok.reshape(NW, DNCH, DCH))
    y_pad = _mlp_call(gate_pad, disp, W1, W2)
    out = combine(y_pad, rowidx.reshape(NW, CNCH, CROWS))
    return out
